# baseline (speedup 1.0000x reference)
import jax
import jax.numpy as jnp
import numpy as np
from jax.experimental import pallas as pl

_N, _E, _L, _DH, _HEADS, _HD, _B = 10000, 160000, 6, 256, 8, 32, 64
_BN = 2000
_BE = 2000
_EPS = 1e-5


def _ln(x, g, b):
    mu = jnp.mean(x, axis=-1, keepdims=True)
    var = jnp.mean((x - mu) ** 2, axis=-1, keepdims=True)
    return (x - mu) * jax.lax.rsqrt(var + _EPS) * g + b


def _silu(x):
    return x * jax.nn.sigmoid(x)


def _dot(a, b):
    return jnp.dot(a, b, preferred_element_type=jnp.float32,
                   precision=jax.lax.Precision.HIGHEST)


def _embed_body(xc, W, pb, g, bb, o):
    y = _dot(xc[...], W[...]) + pb[...]
    o[...] = _silu(_ln(y, g[...], bb[...]))


def _embed(xc, W, pb, g, bb):
    full = lambda r, c: pl.BlockSpec((r, c), lambda i: (0, 0))
    return pl.pallas_call(
        _embed_body,
        grid=(_N // _BN,),
        in_specs=[pl.BlockSpec((_BN, 256), lambda i: (i, 0)),
                  full(256, 256), full(1, 256), full(1, 256), full(1, 256)],
        out_specs=pl.BlockSpec((_BN, 256), lambda i: (i, 0)),
        out_shape=jax.ShapeDtypeStruct((_N, 256), jnp.float32),
    )(xc, W, pb, g, bb)


def _ea_body(d, c, w, o):
    dv = d[...]
    cw = c[...]
    ww = w[...]
    gamma = 1.0 / (2.0 * ww * ww)
    rbf = jnp.exp(-gamma * (dv - cw) ** 2)
    cut = 0.5 * (jnp.cos(np.pi * dv / 10.0) + 1.0) * (dv < 10.0).astype(jnp.float32)
    o[...] = rbf * cut


def _edge_attr(d2, c, w):
    full = lambda r, c_: pl.BlockSpec((r, c_), lambda i: (0, 0))
    return pl.pallas_call(
        _ea_body,
        grid=(_E // _BE,),
        in_specs=[pl.BlockSpec((_BE, 1), lambda i: (i, 0)),
                  full(1, 256), full(1, 256)],
        out_specs=pl.BlockSpec((_BE, 256), lambda i: (i, 0)),
        out_shape=jax.ShapeDtypeStruct((_E, 256), jnp.float32),
    )(d2, c, w)


def _qkvs_body(x, Wq, Wk, Wv, Ws, bq, bk, bv, bs, g, b, qo, ko, vo, so):
    h = _ln(x[...], g[...], b[...])
    qo[...] = _dot(h, Wq[...]) + bq[...]
    ko[...] = _dot(h, Wk[...]) + bk[...]
    vo[...] = _dot(h, Wv[...]) + bv[...]
    so[...] = _dot(h, Ws[...]) + bs[...]


def _qkvs(x, Wq, Wk, Wv, Ws, bq, bk, bv, bs, g, b):
    full = lambda r, c: pl.BlockSpec((r, c), lambda i: (0, 0))
    blk = pl.BlockSpec((_BN, 256), lambda i: (i, 0))
    sh = jax.ShapeDtypeStruct((_N, 256), jnp.float32)
    return pl.pallas_call(
        _qkvs_body,
        grid=(_N // _BN,),
        in_specs=[blk] + [full(256, 256)] * 4 + [full(1, 256)] * 6,
        out_specs=[blk] * 4,
        out_shape=[sh] * 4,
    )(x, Wq, Wk, Wv, Ws, bq, bk, bv, bs, g, b)


def _mm_body(a, W, o):
    o[...] = _dot(a[...], W[...])


def _emm(ea, We):
    return pl.pallas_call(
        _mm_body,
        grid=(_E // _BE,),
        in_specs=[pl.BlockSpec((_BE, 256), lambda i: (i, 0)),
                  pl.BlockSpec((256, 256), lambda i: (0, 0))],
        out_specs=pl.BlockSpec((_BE, 256), lambda i: (i, 0)),
        out_shape=jax.ShapeDtypeStruct((_E, 256), jnp.float32),
    )(ea, We)


def _edgeC_body(ea, We, qd, kvs, G, GT, o):
    e = _dot(ea[...], We[...])
    kvv = kvs[...]
    z = qd[...] * (kvv[:, :_DH] + e)
    s = _dot(z, G[...]) * (1.0 / np.sqrt(float(_HD)))
    aexp = jnp.exp(s)
    abig = _dot(aexp, GT[...])
    o[:, :_DH] = (kvv[:, _DH:] + e) * abig
    o[:, _DH:] = aexp


def _edgeC(ea, We, qd, kvs, G, GT):
    full = lambda r, c: pl.BlockSpec((r, c), lambda i: (0, 0))
    ne = ea.shape[0]
    return pl.pallas_call(
        _edgeC_body,
        grid=(ne // _BE,),
        in_specs=[pl.BlockSpec((_BE, 256), lambda i: (i, 0)),
                  full(256, 256),
                  pl.BlockSpec((_BE, 256), lambda i: (i, 0)),
                  pl.BlockSpec((_BE, 512), lambda i: (i, 0)),
                  full(256, _HEADS),
                  full(_HEADS, 256)],
        out_specs=pl.BlockSpec((_BE, _DH + _HEADS), lambda i: (i, 0)),
        out_shape=jax.ShapeDtypeStruct((ne, _DH + _HEADS), jnp.float32),
    )(ea, We, qd, kvs, G, GT)


def _edgeA_body(ea, We, qd, kvs, G, so):
    e = _dot(ea[...], We[...])
    z = qd[...] * (kvs[...][:, :_DH] + e)
    so[...] = _dot(z, G[...]) * (1.0 / np.sqrt(float(_HD)))


def _edgeA(ea, We, qd, kvs, G):
    full = lambda r, c: pl.BlockSpec((r, c), lambda i: (0, 0))
    return pl.pallas_call(
        _edgeA_body,
        grid=(_E // _BE,),
        in_specs=[pl.BlockSpec((_BE, 256), lambda i: (i, 0)),
                  full(256, 256),
                  pl.BlockSpec((_BE, 256), lambda i: (i, 0)),
                  pl.BlockSpec((_BE, 512), lambda i: (i, 0)),
                  full(256, _HEADS)],
        out_specs=pl.BlockSpec((_BE, _HEADS), lambda i: (i, 0)),
        out_shape=jax.ShapeDtypeStruct((_E, _HEADS), jnp.float32),
    )(ea, We, qd, kvs, G)


def _edgeB_body(ea, We, kvs, s, m, GT, o):
    e = _dot(ea[...], We[...])
    aexp = jnp.exp(s[...] - m[...])
    abig = _dot(aexp, GT[...])
    o[:, :_DH] = (kvs[...][:, _DH:] + e) * abig
    o[:, _DH:] = aexp


def _edgeB(ea, We, kvs, s, m, GT):
    full = lambda r, c: pl.BlockSpec((r, c), lambda i: (0, 0))
    return pl.pallas_call(
        _edgeB_body,
        grid=(_E // _BE,),
        in_specs=[pl.BlockSpec((_BE, 256), lambda i: (i, 0)),
                  full(256, 256),
                  pl.BlockSpec((_BE, 512), lambda i: (i, 0)),
                  pl.BlockSpec((_BE, _HEADS), lambda i: (i, 0)),
                  full(1, _HEADS),
                  full(_HEADS, 256)],
        out_specs=pl.BlockSpec((_BE, _DH + _HEADS), lambda i: (i, 0)),
        out_shape=jax.ShapeDtypeStruct((_E, _DH + _HEADS), jnp.float32),
    )(ea, We, kvs, s, m, GT)


def _ffn_body(x, acc, skip, GT, al, g, b, W1, b1, W2, b2, o):
    den = _dot(acc[...][:, _DH:], GT[...])
    seg = acc[...][:, :_DH] / (den + 1e-16)
    x1 = x[...] + al[0, 0] * (seg + skip[...])
    h = _ln(x1, g[...], b[...])
    h = _dot(_silu(_dot(h, W1[...]) + b1[...]), W2[...]) + b2[...]
    o[...] = x1 + h


def _ffn(x, acc, skip, GT, al, g, b, W1, b1, W2, b2):
    full = lambda r, c: pl.BlockSpec((r, c), lambda i: (0, 0))
    blk = pl.BlockSpec((_BN, 256), lambda i: (i, 0))
    return pl.pallas_call(
        _ffn_body,
        grid=(_N // _BN,),
        in_specs=[blk, pl.BlockSpec((_BN, _DH + _HEADS), lambda i: (i, 0)),
                  blk, full(_HEADS, 256), full(1, 1), full(1, 256), full(1, 256),
                  full(256, 1024), full(1, 1024), full(1024, 256), full(1, 256)],
        out_specs=blk,
        out_shape=jax.ShapeDtypeStruct((_N, 256), jnp.float32),
    )(x, acc, skip, GT, al, g, b, W1, b1, W2, b2)


def _head_body(x4, x5, x6, eW1, eb1, eW2, eb2, fW1, fb1, fW2, fb2,
               xm_o, ae_o, f_o):
    xm = (x4[...] + x5[...] + x6[...]) * (1.0 / 3.0)
    xm_o[...] = xm
    ae_o[...] = _dot(_silu(_dot(xm, eW1[...]) + eb1[...]), eW2[...]) + eb2[...]
    f_o[...] = _dot(_silu(_dot(xm, fW1[...]) + fb1[...]), fW2[...]) + fb2[...]


def _head(x4, x5, x6, eW1, eb1, eW2, eb2, fW1, fb1, fW2, fb2):
    full = lambda r, c: pl.BlockSpec((r, c), lambda i: (0, 0))
    blk = pl.BlockSpec((_BN, 256), lambda i: (i, 0))
    blk128 = pl.BlockSpec((_BN, 128), lambda i: (i, 0))
    return pl.pallas_call(
        _head_body,
        grid=(_N // _BN,),
        in_specs=[blk, blk, blk,
                  full(256, 256), full(1, 256), full(256, 128), full(1, 128),
                  full(256, 256), full(1, 256), full(256, 128), full(1, 128)],
        out_specs=[blk, blk128, blk128],
        out_shape=[jax.ShapeDtypeStruct((_N, 256), jnp.float32),
                   jax.ShapeDtypeStruct((_N, 128), jnp.float32),
                   jax.ShapeDtypeStruct((_N, 128), jnp.float32)],
    )(x4, x5, x6, eW1, eb1, eW2, eb2, fW1, fb1, fW2, fb2)


def kernel(atomic_numbers, pos, edge_index, batch, params):
    p = params
    an = atomic_numbers
    r2 = lambda a: a.reshape(1, -1)

    xc = jnp.concatenate([p['elem_emb'][an], p['radius_emb'][an],
                          p['en_emb'][an], p['ie_emb'][an]], axis=-1)
    xc = jnp.pad(xc, ((0, 0), (0, 125)))
    Wp = jnp.pad(p['proj_W'], ((0, 125), (0, 0)))
    x = _embed(xc, Wp, r2(p['proj_b']), r2(p['proj_ln_g']), r2(p['proj_ln_b']))

    src, dst = edge_index[0], edge_index[1]
    ev = pos[src] - pos[dst]
    d = jnp.sqrt(jnp.sum(ev * ev, axis=-1))
    ea = _edge_attr(d[:, None], r2(p['rbf_centers']), r2(p['rbf_widths']))

    G = jnp.asarray(np.repeat(np.eye(_HEADS, dtype=np.float32), _HD, axis=0))
    GT = jnp.asarray(np.repeat(np.eye(_HEADS, dtype=np.float32), _HD, axis=1))

    feats = [x]
    for l in range(_L):
        q, k, v, skip = _qkvs(x, p['Wq'][l], p['Wk'][l], p['Wv'][l], p['Wskip'][l],
                              r2(p['bq'][l]), r2(p['bk'][l]), r2(p['bv'][l]),
                              r2(p['bskip'][l]), r2(p['n1_g'][l]), r2(p['n1_b'][l]))
        kv = jnp.concatenate([k, v], axis=1)
        acc = None
        for sl in (slice(i * (_E // 4), (i + 1) * (_E // 4)) for i in range(4)):
            kvs = jnp.take(kv, src[sl], axis=0)
            qd = jnp.take(q, dst[sl], axis=0)
            upd = _edgeC(ea[sl], p['We'][l], qd, kvs, G, GT)
            a1 = jax.ops.segment_sum(upd, dst[sl], num_segments=_N)
            acc = a1 if acc is None else acc + a1
        x = _ffn(x, acc, skip, GT, p['alpha'][l].reshape(1, 1),
                 r2(p['n2_g'][l]), r2(p['n2_b'][l]),
                 p['f_W1'][l], r2(p['f_b1'][l]), p['f_W2'][l], r2(p['f_b2'][l]))
        feats.append(x)

    eW2 = jnp.pad(p['e_W2'], ((0, 0), (0, 127)))
    eb2 = jnp.pad(p['e_b2'].reshape(1, 1), ((0, 0), (0, 127)))
    fW2 = jnp.pad(p['fr_W2'], ((0, 0), (0, 125)))
    fb2 = jnp.pad(p['fr_b2'].reshape(1, 3), ((0, 0), (0, 125)))
    xm, aep, fp = _head(feats[-3], feats[-2], feats[-1],
                        p['e_W1'], r2(p['e_b1']), eW2, eb2,
                        p['fr_W1'], r2(p['fr_b1']), fW2, fb2)
    energy = jax.ops.segment_sum(aep[:, :1], batch, num_segments=_B)[:, 0]
    forces = fp[:, :3]
    return energy, forces, xm


# speedup vs baseline: 1.2271x; 1.2271x over previous
import jax
import jax.numpy as jnp
import numpy as np
from jax.experimental import pallas as pl

_N, _E, _L, _DH, _HEADS, _HD, _B = 10000, 160000, 6, 256, 8, 32, 64
_BN = 2000
_BE = 2000
_EPS = 1e-5


def _ln(x, g, b):
    mu = jnp.mean(x, axis=-1, keepdims=True)
    var = jnp.mean((x - mu) ** 2, axis=-1, keepdims=True)
    return (x - mu) * jax.lax.rsqrt(var + _EPS) * g + b


def _silu(x):
    return x * jax.nn.sigmoid(x)


def _dot(a, b):
    return jnp.dot(a, b, preferred_element_type=jnp.float32)


def _embed_body(xc, W, pb, g, bb, o):
    y = _dot(xc[...], W[...]) + pb[...]
    o[...] = _silu(_ln(y, g[...], bb[...]))


def _embed(xc, W, pb, g, bb):
    full = lambda r, c: pl.BlockSpec((r, c), lambda i: (0, 0))
    return pl.pallas_call(
        _embed_body,
        grid=(_N // _BN,),
        in_specs=[pl.BlockSpec((_BN, 256), lambda i: (i, 0)),
                  full(256, 256), full(1, 256), full(1, 256), full(1, 256)],
        out_specs=pl.BlockSpec((_BN, 256), lambda i: (i, 0)),
        out_shape=jax.ShapeDtypeStruct((_N, 256), jnp.float32),
    )(xc, W, pb, g, bb)


def _ea_body(d, c, w, o):
    dv = d[...]
    cw = c[...]
    ww = w[...]
    gamma = 1.0 / (2.0 * ww * ww)
    rbf = jnp.exp(-gamma * (dv - cw) ** 2)
    cut = 0.5 * (jnp.cos(np.pi * dv / 10.0) + 1.0) * (dv < 10.0).astype(jnp.float32)
    o[...] = rbf * cut


def _edge_attr(d2, c, w):
    full = lambda r, c_: pl.BlockSpec((r, c_), lambda i: (0, 0))
    return pl.pallas_call(
        _ea_body,
        grid=(_E // _BE,),
        in_specs=[pl.BlockSpec((_BE, 1), lambda i: (i, 0)),
                  full(1, 256), full(1, 256)],
        out_specs=pl.BlockSpec((_BE, 256), lambda i: (i, 0)),
        out_shape=jax.ShapeDtypeStruct((_E, 256), jnp.float32),
    )(d2, c, w)


def _qkvs_body(x, Wq, Wk, Wv, Ws, bq, bk, bv, bs, g, b, qo, ko, vo, so):
    h = _ln(x[...], g[...], b[...])
    qo[...] = _dot(h, Wq[...]) + bq[...]
    ko[...] = _dot(h, Wk[...]) + bk[...]
    vo[...] = _dot(h, Wv[...]) + bv[...]
    so[...] = _dot(h, Ws[...]) + bs[...]


def _qkvs(x, Wq, Wk, Wv, Ws, bq, bk, bv, bs, g, b):
    full = lambda r, c: pl.BlockSpec((r, c), lambda i: (0, 0))
    blk = pl.BlockSpec((_BN, 256), lambda i: (i, 0))
    sh = jax.ShapeDtypeStruct((_N, 256), jnp.float32)
    return pl.pallas_call(
        _qkvs_body,
        grid=(_N // _BN,),
        in_specs=[blk] + [full(256, 256)] * 4 + [full(1, 256)] * 6,
        out_specs=[blk] * 4,
        out_shape=[sh] * 4,
    )(x, Wq, Wk, Wv, Ws, bq, bk, bv, bs, g, b)


def _mm_body(a, W, o):
    o[...] = _dot(a[...], W[...])


def _emm(ea, We):
    return pl.pallas_call(
        _mm_body,
        grid=(_E // _BE,),
        in_specs=[pl.BlockSpec((_BE, 256), lambda i: (i, 0)),
                  pl.BlockSpec((256, 256), lambda i: (0, 0))],
        out_specs=pl.BlockSpec((_BE, 256), lambda i: (i, 0)),
        out_shape=jax.ShapeDtypeStruct((_E, 256), jnp.float32),
    )(ea, We)


def _edgeC_body(ea, We, qd, kvs, G, GT, o):
    e = _dot(ea[...], We[...])
    kvv = kvs[...]
    z = qd[...] * (kvv[:, :_DH] + e)
    s = _dot(z, G[...]) * (1.0 / np.sqrt(float(_HD)))
    aexp = jnp.exp(jnp.minimum(s, 80.0))
    abig = _dot(aexp, GT[...])
    o[:, :_DH] = (kvv[:, _DH:] + e) * abig
    o[:, _DH:] = aexp


def _edgeC(ea, We, qd, kvs, G, GT):
    full = lambda r, c: pl.BlockSpec((r, c), lambda i: (0, 0))
    ne = ea.shape[0]
    return pl.pallas_call(
        _edgeC_body,
        grid=(ne // _BE,),
        in_specs=[pl.BlockSpec((_BE, 256), lambda i: (i, 0)),
                  full(256, 256),
                  pl.BlockSpec((_BE, 256), lambda i: (i, 0)),
                  pl.BlockSpec((_BE, 512), lambda i: (i, 0)),
                  full(256, _HEADS),
                  full(_HEADS, 256)],
        out_specs=pl.BlockSpec((_BE, _DH + _HEADS), lambda i: (i, 0)),
        out_shape=jax.ShapeDtypeStruct((ne, _DH + _HEADS), jnp.float32),
    )(ea, We, qd, kvs, G, GT)


def _edgeA_body(ea, We, qd, kvs, G, so):
    e = _dot(ea[...], We[...])
    z = qd[...] * (kvs[...][:, :_DH] + e)
    so[...] = _dot(z, G[...]) * (1.0 / np.sqrt(float(_HD)))


def _edgeA(ea, We, qd, kvs, G):
    full = lambda r, c: pl.BlockSpec((r, c), lambda i: (0, 0))
    return pl.pallas_call(
        _edgeA_body,
        grid=(_E // _BE,),
        in_specs=[pl.BlockSpec((_BE, 256), lambda i: (i, 0)),
                  full(256, 256),
                  pl.BlockSpec((_BE, 256), lambda i: (i, 0)),
                  pl.BlockSpec((_BE, 512), lambda i: (i, 0)),
                  full(256, _HEADS)],
        out_specs=pl.BlockSpec((_BE, _HEADS), lambda i: (i, 0)),
        out_shape=jax.ShapeDtypeStruct((_E, _HEADS), jnp.float32),
    )(ea, We, qd, kvs, G)


def _edgeB_body(ea, We, kvs, s, m, GT, o):
    e = _dot(ea[...], We[...])
    aexp = jnp.exp(s[...] - m[...])
    abig = _dot(aexp, GT[...])
    o[:, :_DH] = (kvs[...][:, _DH:] + e) * abig
    o[:, _DH:] = aexp


def _edgeB(ea, We, kvs, s, m, GT):
    full = lambda r, c: pl.BlockSpec((r, c), lambda i: (0, 0))
    return pl.pallas_call(
        _edgeB_body,
        grid=(_E // _BE,),
        in_specs=[pl.BlockSpec((_BE, 256), lambda i: (i, 0)),
                  full(256, 256),
                  pl.BlockSpec((_BE, 512), lambda i: (i, 0)),
                  pl.BlockSpec((_BE, _HEADS), lambda i: (i, 0)),
                  full(1, _HEADS),
                  full(_HEADS, 256)],
        out_specs=pl.BlockSpec((_BE, _DH + _HEADS), lambda i: (i, 0)),
        out_shape=jax.ShapeDtypeStruct((_E, _DH + _HEADS), jnp.float32),
    )(ea, We, kvs, s, m, GT)


def _ffn_body(x, acc, skip, GT, al, g, b, W1, b1, W2, b2, o):
    den = _dot(acc[...][:, _DH:], GT[...])
    seg = acc[...][:, :_DH] / (den + 1e-16)
    x1 = x[...] + al[0, 0] * (seg + skip[...])
    h = _ln(x1, g[...], b[...])
    h = _dot(_silu(_dot(h, W1[...]) + b1[...]), W2[...]) + b2[...]
    o[...] = x1 + h


def _ffn(x, acc, skip, GT, al, g, b, W1, b1, W2, b2):
    full = lambda r, c: pl.BlockSpec((r, c), lambda i: (0, 0))
    blk = pl.BlockSpec((_BN, 256), lambda i: (i, 0))
    return pl.pallas_call(
        _ffn_body,
        grid=(_N // _BN,),
        in_specs=[blk, pl.BlockSpec((_BN, _DH + _HEADS), lambda i: (i, 0)),
                  blk, full(_HEADS, 256), full(1, 1), full(1, 256), full(1, 256),
                  full(256, 1024), full(1, 1024), full(1024, 256), full(1, 256)],
        out_specs=blk,
        out_shape=jax.ShapeDtypeStruct((_N, 256), jnp.float32),
    )(x, acc, skip, GT, al, g, b, W1, b1, W2, b2)


def _head_body(x4, x5, x6, eW1, eb1, eW2, eb2, fW1, fb1, fW2, fb2,
               xm_o, ae_o, f_o):
    xm = (x4[...] + x5[...] + x6[...]) * (1.0 / 3.0)
    xm_o[...] = xm
    ae_o[...] = _dot(_silu(_dot(xm, eW1[...]) + eb1[...]), eW2[...]) + eb2[...]
    f_o[...] = _dot(_silu(_dot(xm, fW1[...]) + fb1[...]), fW2[...]) + fb2[...]


def _head(x4, x5, x6, eW1, eb1, eW2, eb2, fW1, fb1, fW2, fb2):
    full = lambda r, c: pl.BlockSpec((r, c), lambda i: (0, 0))
    blk = pl.BlockSpec((_BN, 256), lambda i: (i, 0))
    blk128 = pl.BlockSpec((_BN, 128), lambda i: (i, 0))
    return pl.pallas_call(
        _head_body,
        grid=(_N // _BN,),
        in_specs=[blk, blk, blk,
                  full(256, 256), full(1, 256), full(256, 128), full(1, 128),
                  full(256, 256), full(1, 256), full(256, 128), full(1, 128)],
        out_specs=[blk, blk128, blk128],
        out_shape=[jax.ShapeDtypeStruct((_N, 256), jnp.float32),
                   jax.ShapeDtypeStruct((_N, 128), jnp.float32),
                   jax.ShapeDtypeStruct((_N, 128), jnp.float32)],
    )(x4, x5, x6, eW1, eb1, eW2, eb2, fW1, fb1, fW2, fb2)


def kernel(atomic_numbers, pos, edge_index, batch, params):
    p = params
    an = atomic_numbers
    r2 = lambda a: a.reshape(1, -1)

    xc = jnp.concatenate([p['elem_emb'][an], p['radius_emb'][an],
                          p['en_emb'][an], p['ie_emb'][an]], axis=-1)
    xc = jnp.pad(xc, ((0, 0), (0, 125)))
    Wp = jnp.pad(p['proj_W'], ((0, 125), (0, 0)))
    x = _embed(xc, Wp, r2(p['proj_b']), r2(p['proj_ln_g']), r2(p['proj_ln_b']))

    src, dst = edge_index[0], edge_index[1]
    ev = pos[src] - pos[dst]
    d = jnp.sqrt(jnp.sum(ev * ev, axis=-1))
    ea = _edge_attr(d[:, None], r2(p['rbf_centers']), r2(p['rbf_widths']))

    G = jnp.asarray(np.repeat(np.eye(_HEADS, dtype=np.float32), _HD, axis=0))
    GT = jnp.asarray(np.repeat(np.eye(_HEADS, dtype=np.float32), _HD, axis=1))

    feats = [x]
    for l in range(_L):
        q, k, v, skip = _qkvs(x, p['Wq'][l], p['Wk'][l], p['Wv'][l], p['Wskip'][l],
                              r2(p['bq'][l]), r2(p['bk'][l]), r2(p['bv'][l]),
                              r2(p['bskip'][l]), r2(p['n1_g'][l]), r2(p['n1_b'][l]))
        kv = jnp.concatenate([k, v], axis=1)
        acc = None
        for sl in (slice(i * (_E // 4), (i + 1) * (_E // 4)) for i in range(4)):
            kvs = jnp.take(kv, src[sl], axis=0)
            qd = jnp.take(q, dst[sl], axis=0)
            upd = _edgeC(ea[sl], p['We'][l], qd, kvs, G, GT)
            a1 = jax.ops.segment_sum(upd, dst[sl], num_segments=_N)
            acc = a1 if acc is None else acc + a1
        x = _ffn(x, acc, skip, GT, p['alpha'][l].reshape(1, 1),
                 r2(p['n2_g'][l]), r2(p['n2_b'][l]),
                 p['f_W1'][l], r2(p['f_b1'][l]), p['f_W2'][l], r2(p['f_b2'][l]))
        feats.append(x)

    eW2 = jnp.pad(p['e_W2'], ((0, 0), (0, 127)))
    eb2 = jnp.pad(p['e_b2'].reshape(1, 1), ((0, 0), (0, 127)))
    fW2 = jnp.pad(p['fr_W2'], ((0, 0), (0, 125)))
    fb2 = jnp.pad(p['fr_b2'].reshape(1, 3), ((0, 0), (0, 125)))
    xm, aep, fp = _head(feats[-3], feats[-2], feats[-1],
                        p['e_W1'], r2(p['e_b1']), eW2, eb2,
                        p['fr_W1'], r2(p['fr_b1']), fW2, fb2)
    energy = jax.ops.segment_sum(aep[:, :1], batch, num_segments=_B)[:, 0]
    forces = fp[:, :3]
    return energy, forces, xm
